# baseline (speedup 1.0000x reference)
import functools

import jax
import jax.numpy as jnp
from jax.experimental import pallas as pl
from jax.experimental.pallas import tpu as pltpu

_HIGH = jax.lax.Precision.HIGHEST


def _gelu(v):
    return 0.5 * v * (1.0 + jax.lax.erf(v * 0.7071067811865476))


def _dot(a, b):
    return jax.lax.dot_general(a, b, (((1,), (0,)), ((), ())),
                               precision=_HIGH,
                               preferred_element_type=jnp.float32)


def _proj_body(x_ref, w_ref, b_ref, o_ref):
    o_ref[...] = _dot(x_ref[...], w_ref[...]) + b_ref[...]


def _project(x, wcat, bcat, blk_n):
    n, d = x.shape
    h2 = wcat.shape[1]
    grid = (n // blk_n,)
    return pl.pallas_call(
        _proj_body,
        grid=grid,
        in_specs=[
            pl.BlockSpec((blk_n, d), lambda i: (i, 0)),
            pl.BlockSpec((d, h2), lambda i: (0, 0)),
            pl.BlockSpec((1, h2), lambda i: (0, 0)),
        ],
        out_specs=pl.BlockSpec((blk_n, h2), lambda i: (i, 0)),
        out_shape=jax.ShapeDtypeStruct((n, h2), jnp.float32),
    )(x, wcat, bcat)


def _chain_body(g_ref, ea_ref, w1e_ref, b1_ref, w2_ref, b2_ref,
                w3_ref, b3_ref, wa_ref, ba_ref, h3_ref, raw_ref):
    h0 = g_ref[...] + _dot(ea_ref[...], w1e_ref[...]) + b1_ref[...]
    h1 = _gelu(h0)
    h2 = _gelu(_dot(h1, w2_ref[...]) + b2_ref[...])
    h3 = _gelu(_dot(h2, w3_ref[...]) + b3_ref[...])
    h3_ref[...] = h3
    r = _dot(h3, wa_ref[...]) + ba_ref[...]
    raw_ref[...] = jnp.where(r >= 0.0, r, 0.1 * r)


def _edge_chain(g, ea, w1e, b1, w2, b2, w3, b3, wa, ba, blk_e):
    e = g.shape[0]
    h = g.shape[1]
    de = ea.shape[1]
    grid = (e // blk_e,)
    h3, raw = pl.pallas_call(
        _chain_body,
        grid=grid,
        in_specs=[
            pl.BlockSpec((blk_e, h), lambda i: (i, 0)),
            pl.BlockSpec((blk_e, de), lambda i: (i, 0)),
            pl.BlockSpec((de, h), lambda i: (0, 0)),
            pl.BlockSpec((1, h), lambda i: (0, 0)),
            pl.BlockSpec((h, h), lambda i: (0, 0)),
            pl.BlockSpec((1, h), lambda i: (0, 0)),
            pl.BlockSpec((h, h), lambda i: (0, 0)),
            pl.BlockSpec((1, h), lambda i: (0, 0)),
            pl.BlockSpec((h, 1), lambda i: (0, 0)),
            pl.BlockSpec((1, 1), lambda i: (0, 0)),
        ],
        out_specs=[
            pl.BlockSpec((blk_e, h), lambda i: (i, 0)),
            pl.BlockSpec((blk_e, 1), lambda i: (i, 0)),
        ],
        out_shape=[
            jax.ShapeDtypeStruct((e, h), jnp.float32),
            jax.ShapeDtypeStruct((e, 1), jnp.float32),
        ],
    )(g, ea, w1e, b1, w2, b2, w3, b3, wa, ba)
    return h3, raw


def _fc_body(x_ref, w1_ref, b1_ref, w2_ref, b2_ref, w3_ref, b3_ref, o_ref):
    h = _gelu(_dot(x_ref[...], w1_ref[...]) + b1_ref[...])
    h = _gelu(_dot(h, w2_ref[...]) + b2_ref[...])
    o_ref[...] = _dot(h, w3_ref[...]) + b3_ref[...]


def _fc_head(x, p1, p2, p3, blk_n):
    n, h = x.shape
    w1, b1 = p1
    w2, b2 = p2
    w3, b3 = p3
    d1 = w1.shape[1]
    d2 = w2.shape[1]
    d3 = w3.shape[1]
    grid = (n // blk_n,)
    return pl.pallas_call(
        _fc_body,
        grid=grid,
        in_specs=[
            pl.BlockSpec((blk_n, h), lambda i: (i, 0)),
            pl.BlockSpec((h, d1), lambda i: (0, 0)),
            pl.BlockSpec((1, d1), lambda i: (0, 0)),
            pl.BlockSpec((d1, d2), lambda i: (0, 0)),
            pl.BlockSpec((1, d2), lambda i: (0, 0)),
            pl.BlockSpec((d2, d3), lambda i: (0, 0)),
            pl.BlockSpec((1, d3), lambda i: (0, 0)),
        ],
        out_specs=pl.BlockSpec((blk_n, d3), lambda i: (i, 0)),
        out_shape=jax.ShapeDtypeStruct((n, d3), jnp.float32),
    )(x, w1, b1[None, :], w2, b2[None, :], w3, b3[None, :])


def _layer(p, node, src, dst, edge_attr, n, blk_n, blk_e):
    in_ch = node.shape[1]
    h = p['W2'].shape[0]
    w1 = p['W1']
    w_dst = w1[:in_ch]
    w_src = w1[in_ch:2 * in_ch]
    w1e = w1[2 * in_ch:]
    wcat = jnp.concatenate([w_dst, w_src], axis=1)
    bcat = jnp.zeros((1, 2 * h), jnp.float32)
    proj = _project(node, wcat, bcat, blk_n)
    g = proj[dst, :h] + proj[src, h:]
    h3, raw = _edge_chain(g, edge_attr, w1e, p['b1'][None, :],
                          p['W2'], p['b2'][None, :],
                          p['W3'], p['b3'][None, :],
                          p['Wa'], p['ba'][None, :], blk_e)
    raw = raw[:, 0]
    m = jax.ops.segment_max(raw, dst, num_segments=n)
    m = jnp.where(jnp.isfinite(m), m, 0.0)
    e_w = jnp.exp(raw - m[dst])
    s = jax.ops.segment_sum(e_w, dst, num_segments=n)
    aggr = jax.ops.segment_sum(e_w[:, None] * h3, dst, num_segments=n)
    aggr = aggr / (s[:, None] + 1e-16)
    return _gelu(aggr)


def kernel(x, edge_index, edge_attr, params):
    n = x.shape[0]
    src = edge_index[0]
    dst = edge_index[1]
    blk_n = 1000
    blk_e = 1000
    node = x
    for name in ('gnn1', 'gnn2', 'gnn3', 'gnn4'):
        node = _layer(params[name], node, src, dst, edge_attr, n, blk_n, blk_e)
    return _fc_head(node, params['fc1'], params['fc2'], params['fc3'], blk_n)


# speedup vs baseline: 72.0096x; 72.0096x over previous
import functools

import jax
import jax.numpy as jnp
from jax import lax
from jax.experimental import pallas as pl
from jax.experimental.pallas import tpu as pltpu
from jax.experimental.pallas import tpu_sc as plsc

_HIGH = jax.lax.Precision.DEFAULT

_NC = 2
_NS = 16
_NW = _NC * _NS


def _gelu(v):
    return 0.5 * v * (1.0 + jax.lax.erf(v * 0.7071067811865476))


def _dot(a, b):
    return jax.lax.dot_general(a, b, (((1,), (0,)), ((), ())),
                               precision=_HIGH,
                               preferred_element_type=jnp.float32)


def _proj_body(x_ref, w_ref, p_ref):
    p_ref[...] = _dot(x_ref[...], w_ref[...])


def _project(x, wcat, blk_n):
    n, d = x.shape
    return pl.pallas_call(
        _proj_body,
        grid=(n // blk_n,),
        in_specs=[
            pl.BlockSpec((blk_n, d), lambda i: (i, 0)),
            pl.BlockSpec((d, 128), lambda i: (0, 0)),
        ],
        out_specs=pl.BlockSpec((blk_n, 128), lambda i: (i, 0)),
        out_shape=jax.ShapeDtypeStruct((n, 128), jnp.float32),
    )(x, wcat)


def _norm_proj_body(acc_ref, w_ref, p_ref):
    a = acc_ref[...]
    s = a[:, 64:65] + 1e-16
    node = _gelu(a[:, :64] / s)
    p_ref[...] = _dot(node, w_ref[...])


def _norm_project(acc, wcat, blk_n):
    n = acc.shape[0]
    return pl.pallas_call(
        _norm_proj_body,
        grid=(n // blk_n,),
        in_specs=[
            pl.BlockSpec((blk_n, 128), lambda i: (i, 0)),
            pl.BlockSpec((64, 128), lambda i: (0, 0)),
        ],
        out_specs=pl.BlockSpec((blk_n, 128), lambda i: (i, 0)),
        out_shape=jax.ShapeDtypeStruct((n, 128), jnp.float32),
    )(acc, wcat)


def _chain_body(gd_ref, gs_ref, ea_ref, w1e_ref, b1_ref, w2_ref, b2_ref,
                w3_ref, b3_ref, wa_ref, ba_ref, pay_ref):
    h0 = gd_ref[:, :64] + gs_ref[:, 64:] + _dot(ea_ref[...], w1e_ref[...]) \
        + b1_ref[...]
    h1 = _gelu(h0)
    h2 = _gelu(_dot(h1, w2_ref[...]) + b2_ref[...])
    h3 = _gelu(_dot(h2, w3_ref[...]) + b3_ref[...])
    r = _dot(h3, wa_ref[...]) + ba_ref[...]
    raw = jnp.where(r >= 0.0, r, 0.1 * r)
    u = jnp.exp(raw)
    blk = h0.shape[0]
    pay_ref[...] = jnp.concatenate(
        [h3 * u, u, jnp.zeros((blk, 63), jnp.float32)], axis=1)


def _edge_chain(gd, gs, ea, w1e, b1, w2, b2, w3, b3, wa, ba, blk_e):
    e = gd.shape[0]
    h = 64
    de = ea.shape[1]
    return pl.pallas_call(
        _chain_body,
        grid=(e // blk_e,),
        in_specs=[
            pl.BlockSpec((blk_e, 128), lambda i: (i, 0)),
            pl.BlockSpec((blk_e, 128), lambda i: (i, 0)),
            pl.BlockSpec((blk_e, de), lambda i: (i, 0)),
            pl.BlockSpec((de, h), lambda i: (0, 0)),
            pl.BlockSpec((1, h), lambda i: (0, 0)),
            pl.BlockSpec((h, h), lambda i: (0, 0)),
            pl.BlockSpec((1, h), lambda i: (0, 0)),
            pl.BlockSpec((h, h), lambda i: (0, 0)),
            pl.BlockSpec((1, h), lambda i: (0, 0)),
            pl.BlockSpec((h, 1), lambda i: (0, 0)),
            pl.BlockSpec((1, 1), lambda i: (0, 0)),
        ],
        out_specs=pl.BlockSpec((blk_e, 128), lambda i: (i, 0)),
        out_shape=jax.ShapeDtypeStruct((e, 128), jnp.float32),
    )(gd, gs, ea, w1e, b1, w2, b2, w3, b3, wa, ba)


def _sc_gather(ptab, dst, src):
    n = ptab.shape[0]
    e = dst.shape[0]
    per_w = e // _NW
    chunk = 400
    nch = per_w // chunk
    mesh = plsc.VectorSubcoreMesh(core_axis_name="c", subcore_axis_name="s")

    @functools.partial(
        pl.kernel,
        mesh=mesh,
        out_type=[
            jax.ShapeDtypeStruct((e, 128), jnp.float32),
            jax.ShapeDtypeStruct((e, 128), jnp.float32),
        ],
        scratch_types=[
            pltpu.VMEM((chunk,), jnp.int32),
            pltpu.VMEM((chunk,), jnp.int32),
            pltpu.VMEM((chunk, 128), jnp.float32),
            pltpu.VMEM((chunk, 128), jnp.float32),
            pltpu.SemaphoreType.DMA,
            pltpu.SemaphoreType.DMA,
        ],
    )
    def k(p_hbm, dst_hbm, src_hbm, gd_hbm, gs_hbm,
          idx_d, idx_s, rows_d, rows_s, sem_d, sem_s):
        wid = lax.axis_index("c") * _NS + lax.axis_index("s")

        @pl.loop(0, nch)
        def _(ch):
            base = wid * per_w + ch * chunk
            pltpu.sync_copy(dst_hbm.at[pl.ds(base, chunk)], idx_d)
            pltpu.sync_copy(src_hbm.at[pl.ds(base, chunk)], idx_s)
            cp_d = pltpu.async_copy(p_hbm.at[idx_d], rows_d, sem_d)
            cp_s = pltpu.async_copy(p_hbm.at[idx_s], rows_s, sem_s)
            cp_d.wait()
            cp_s.wait()
            pltpu.sync_copy(rows_d, gd_hbm.at[pl.ds(base, chunk)])
            pltpu.sync_copy(rows_s, gs_hbm.at[pl.ds(base, chunk)])

    return k(ptab, dst, src)


def _sc_scatter(pay, dst, zeros_rows, n):
    e = dst.shape[0]
    per_w = e // _NS
    chunk = 200
    nch = per_w // chunk
    npad = ((n + 8 * _NS - 1) // (8 * _NS)) * (8 * _NS)
    rows_per_w = npad // _NS
    mesh = plsc.VectorSubcoreMesh(core_axis_name="c", subcore_axis_name="s",
                                  num_cores=1)

    @functools.partial(
        pl.kernel,
        mesh=mesh,
        out_type=jax.ShapeDtypeStruct((npad, 128), jnp.float32),
        scratch_types=[
            pltpu.VMEM((chunk,), jnp.int32),
            pltpu.VMEM((chunk, 128), jnp.float32),
            pltpu.VMEM_SHARED((npad, 128), jnp.float32),
        ],
    )
    def k(pay_hbm, dst_hbm, zero_hbm, acc_hbm, idx_v, pay_v, acc_s):
        sid = lax.axis_index("s")
        row0 = sid * rows_per_w
        pltpu.sync_copy(zero_hbm, acc_s.at[pl.ds(row0, rows_per_w)])
        plsc.subcore_barrier()

        @pl.loop(0, nch)
        def _(ch):
            base = sid * per_w + ch * chunk
            pltpu.sync_copy(dst_hbm.at[pl.ds(base, chunk)], idx_v)
            pltpu.sync_copy(pay_hbm.at[pl.ds(base, chunk)], pay_v)
            pltpu.sync_copy(pay_v, acc_s.at[idx_v], add=True)

        plsc.subcore_barrier()
        pltpu.sync_copy(acc_s.at[pl.ds(row0, rows_per_w)],
                        acc_hbm.at[pl.ds(row0, rows_per_w)])

    return k(pay, dst, zeros_rows)


def _norm_fc_body(acc_ref, w1_ref, b1_ref, w2_ref, b2_ref, w3_ref, b3_ref,
                  o_ref):
    a = acc_ref[...]
    s = a[:, 64:65] + 1e-16
    node = _gelu(a[:, :64] / s)
    h = _gelu(_dot(node, w1_ref[...]) + b1_ref[...])
    h = _gelu(_dot(h, w2_ref[...]) + b2_ref[...])
    o_ref[...] = _dot(h, w3_ref[...]) + b3_ref[...]


def _norm_fc(acc, p1, p2, p3, blk_n):
    n = acc.shape[0]
    w1, b1 = p1
    w2, b2 = p2
    w3, b3 = p3
    d1, d2, d3 = w1.shape[1], w2.shape[1], w3.shape[1]
    return pl.pallas_call(
        _norm_fc_body,
        grid=(n // blk_n,),
        in_specs=[
            pl.BlockSpec((blk_n, 128), lambda i: (i, 0)),
            pl.BlockSpec((64, d1), lambda i: (0, 0)),
            pl.BlockSpec((1, d1), lambda i: (0, 0)),
            pl.BlockSpec((d1, d2), lambda i: (0, 0)),
            pl.BlockSpec((1, d2), lambda i: (0, 0)),
            pl.BlockSpec((d2, d3), lambda i: (0, 0)),
            pl.BlockSpec((1, d3), lambda i: (0, 0)),
        ],
        out_specs=pl.BlockSpec((blk_n, d3), lambda i: (i, 0)),
        out_shape=jax.ShapeDtypeStruct((n, d3), jnp.float32),
    )(acc, w1, b1[None, :], w2, b2[None, :], w3, b3[None, :])


def kernel(x, edge_index, edge_attr, params):
    n = x.shape[0]
    h = params['gnn1']['W2'].shape[0]
    src = edge_index[0]
    dst = edge_index[1]
    blk_n = 1000
    blk_e = 2000
    npad = ((n + 8 * _NS - 1) // (8 * _NS)) * (8 * _NS)
    zeros_rows = jnp.zeros((npad // _NS, 128), jnp.float32)

    def split_w1(p, in_ch):
        w1 = p['W1']
        wcat = jnp.concatenate([w1[:in_ch], w1[in_ch:2 * in_ch]], axis=1)
        return wcat, w1[2 * in_ch:]

    acc = None
    for i, name in enumerate(('gnn1', 'gnn2', 'gnn3', 'gnn4')):
        p = params[name]
        in_ch = x.shape[1] if i == 0 else h
        wcat, w1e = split_w1(p, in_ch)
        if i == 0:
            ptab = _project(x, wcat, blk_n)
        else:
            ptab = _norm_project(acc, wcat, blk_n)
        gd, gs = _sc_gather(ptab, dst, src)
        pay = _edge_chain(gd, gs, edge_attr, w1e, p['b1'][None, :],
                          p['W2'], p['b2'][None, :],
                          p['W3'], p['b3'][None, :],
                          p['Wa'], p['ba'][None, :], blk_e)
        acc = _sc_scatter(pay, dst, zeros_rows, n)[:n]

    return _norm_fc(acc, params['fc1'], params['fc2'], params['fc3'], blk_n)


# speedup vs baseline: 87.0590x; 1.2090x over previous
import functools

import jax
import jax.numpy as jnp
from jax import lax
from jax.experimental import pallas as pl
from jax.experimental.pallas import tpu as pltpu
from jax.experimental.pallas import tpu_sc as plsc

_HIGH = jax.lax.Precision.DEFAULT

_NC = 2
_NS = 16
_NW = _NC * _NS


def _gelu(v):
    return 0.5 * v * (1.0 + jax.lax.erf(v * 0.7071067811865476))


def _dot(a, b):
    return jax.lax.dot_general(a, b, (((1,), (0,)), ((), ())),
                               precision=_HIGH,
                               preferred_element_type=jnp.float32)


def _proj_body(x_ref, w_ref, p_ref):
    p_ref[...] = _dot(x_ref[...], w_ref[...])


def _project(x, wcat, blk_n):
    n, d = x.shape
    return pl.pallas_call(
        _proj_body,
        grid=(n // blk_n,),
        in_specs=[
            pl.BlockSpec((blk_n, d), lambda i: (i, 0)),
            pl.BlockSpec((d, 128), lambda i: (0, 0)),
        ],
        out_specs=pl.BlockSpec((blk_n, 128), lambda i: (i, 0)),
        out_shape=jax.ShapeDtypeStruct((n, 128), jnp.float32),
    )(x, wcat)


def _norm_proj_body(acc_ref, w_ref, p_ref):
    a = acc_ref[...]
    s = a[:, 64:65] + 1e-16
    node = _gelu(a[:, :64] / s)
    p_ref[...] = _dot(node, w_ref[...])


def _norm_project(acc, wcat, blk_n):
    n = acc.shape[0]
    return pl.pallas_call(
        _norm_proj_body,
        grid=(n // blk_n,),
        in_specs=[
            pl.BlockSpec((blk_n, 128), lambda i: (i, 0)),
            pl.BlockSpec((64, 128), lambda i: (0, 0)),
        ],
        out_specs=pl.BlockSpec((blk_n, 128), lambda i: (i, 0)),
        out_shape=jax.ShapeDtypeStruct((n, 128), jnp.float32),
    )(acc, wcat)


def _chain_body(gd_ref, gs_ref, ea_ref, w1e_ref, b1_ref, w2_ref, b2_ref,
                w3_ref, b3_ref, wa_ref, ba_ref, pay_ref):
    h0 = gd_ref[:, :64] + gs_ref[:, 64:] + _dot(ea_ref[...], w1e_ref[...]) \
        + b1_ref[...]
    h1 = _gelu(h0)
    h2 = _gelu(_dot(h1, w2_ref[...]) + b2_ref[...])
    h3 = _gelu(_dot(h2, w3_ref[...]) + b3_ref[...])
    r = _dot(h3, wa_ref[...]) + ba_ref[...]
    raw = jnp.where(r >= 0.0, r, 0.1 * r)
    u = jnp.exp(raw)
    blk = h0.shape[0]
    pay_ref[...] = jnp.concatenate(
        [h3 * u, u, jnp.zeros((blk, 63), jnp.float32)], axis=1)


def _edge_chain(gd, gs, ea, w1e, b1, w2, b2, w3, b3, wa, ba, blk_e):
    e = gd.shape[0]
    h = 64
    de = ea.shape[1]
    return pl.pallas_call(
        _chain_body,
        grid=(e // blk_e,),
        in_specs=[
            pl.BlockSpec((blk_e, 128), lambda i: (i, 0)),
            pl.BlockSpec((blk_e, 128), lambda i: (i, 0)),
            pl.BlockSpec((blk_e, de), lambda i: (i, 0)),
            pl.BlockSpec((de, h), lambda i: (0, 0)),
            pl.BlockSpec((1, h), lambda i: (0, 0)),
            pl.BlockSpec((h, h), lambda i: (0, 0)),
            pl.BlockSpec((1, h), lambda i: (0, 0)),
            pl.BlockSpec((h, h), lambda i: (0, 0)),
            pl.BlockSpec((1, h), lambda i: (0, 0)),
            pl.BlockSpec((h, 1), lambda i: (0, 0)),
            pl.BlockSpec((1, 1), lambda i: (0, 0)),
        ],
        out_specs=pl.BlockSpec((blk_e, 128), lambda i: (i, 0)),
        out_shape=jax.ShapeDtypeStruct((e, 128), jnp.float32),
    )(gd, gs, ea, w1e, b1, w2, b2, w3, b3, wa, ba)


def _sc_gather(ptab, dst, src):
    n = ptab.shape[0]
    e = dst.shape[0]
    per_w = e // _NW
    chunk = 200
    nch = per_w // chunk
    mesh = plsc.VectorSubcoreMesh(core_axis_name="c", subcore_axis_name="s")

    @functools.partial(
        pl.kernel,
        mesh=mesh,
        out_type=[
            jax.ShapeDtypeStruct((e, 128), jnp.float32),
            jax.ShapeDtypeStruct((e, 128), jnp.float32),
        ],
        scratch_types=[
            pltpu.VMEM((chunk,), jnp.int32),
            pltpu.VMEM((chunk,), jnp.int32),
            pltpu.VMEM((chunk,), jnp.int32),
            pltpu.VMEM((chunk,), jnp.int32),
            pltpu.VMEM((2, chunk, 128), jnp.float32),
            pltpu.VMEM((2, chunk, 128), jnp.float32),
            pltpu.SemaphoreType.DMA,
            pltpu.SemaphoreType.DMA,
            pltpu.SemaphoreType.DMA,
            pltpu.SemaphoreType.DMA,
            pltpu.SemaphoreType.DMA,
            pltpu.SemaphoreType.DMA,
            pltpu.SemaphoreType.DMA,
            pltpu.SemaphoreType.DMA,
        ],
    )
    def k(p_hbm, dst_hbm, src_hbm, gd_hbm, gs_hbm,
          idx_d0, idx_d1, idx_s0, idx_s1, rows_d, rows_s,
          gsem_d0, gsem_s0, gsem_d1, gsem_s1,
          wsem_d0, wsem_s0, wsem_d1, wsem_s1):
        wid = lax.axis_index("c") * _NS + lax.axis_index("s")
        idxs_d = (idx_d0, idx_d1)
        idxs_s = (idx_s0, idx_s1)
        gsems = ((gsem_d0, gsem_s0), (gsem_d1, gsem_s1))
        wsems = ((wsem_d0, wsem_s0), (wsem_d1, wsem_s1))

        def issue(ch, b):
            base = wid * per_w + ch * chunk
            pltpu.sync_copy(dst_hbm.at[pl.ds(base, chunk)], idxs_d[b])
            pltpu.sync_copy(src_hbm.at[pl.ds(base, chunk)], idxs_s[b])
            pltpu.async_copy(p_hbm.at[idxs_d[b]], rows_d.at[b], gsems[b][0])
            pltpu.async_copy(p_hbm.at[idxs_s[b]], rows_s.at[b], gsems[b][1])

        def gwait(ch, b):
            pltpu.make_async_copy(p_hbm.at[idxs_d[b]], rows_d.at[b],
                                  gsems[b][0]).wait()
            pltpu.make_async_copy(p_hbm.at[idxs_s[b]], rows_s.at[b],
                                  gsems[b][1]).wait()

        def wb(ch, b):
            base = wid * per_w + ch * chunk
            pltpu.async_copy(rows_d.at[b], gd_hbm.at[pl.ds(base, chunk)],
                             wsems[b][0])
            pltpu.async_copy(rows_s.at[b], gs_hbm.at[pl.ds(base, chunk)],
                             wsems[b][1])

        def wbwait(ch, b):
            base = wid * per_w + ch * chunk
            pltpu.make_async_copy(rows_d.at[b], gd_hbm.at[pl.ds(base, chunk)],
                                  wsems[b][0]).wait()
            pltpu.make_async_copy(rows_s.at[b], gs_hbm.at[pl.ds(base, chunk)],
                                  wsems[b][1]).wait()

        issue(0, 0)
        issue(1, 1)

        @pl.loop(0, nch, step=2)
        def _(i):
            gwait(i, 0)
            wb(i, 0)

            @pl.when(i + 2 < nch)
            def _():
                wbwait(i, 0)
                issue(i + 2, 0)

            @pl.when(i + 1 < nch)
            def _():
                gwait(i + 1, 1)
                wb(i + 1, 1)

            @pl.when(i + 3 < nch)
            def _():
                wbwait(i + 1, 1)
                issue(i + 3, 1)

        wbwait(nch - 2, 0)
        wbwait(nch - 1, 1)

    return k(ptab, dst, src)


def _sc_scatter(pay, dst, zeros_rows, n):
    e = dst.shape[0]
    per_w = e // _NS
    chunk = 160
    nch = per_w // chunk
    npad = ((n + 8 * _NS - 1) // (8 * _NS)) * (8 * _NS)
    rows_per_w = npad // _NS
    mesh = plsc.VectorSubcoreMesh(core_axis_name="c", subcore_axis_name="s",
                                  num_cores=1)

    @functools.partial(
        pl.kernel,
        mesh=mesh,
        out_type=jax.ShapeDtypeStruct((npad, 128), jnp.float32),
        scratch_types=[
            pltpu.VMEM((chunk,), jnp.int32),
            pltpu.VMEM((chunk,), jnp.int32),
            pltpu.VMEM((2, chunk, 128), jnp.float32),
            pltpu.VMEM_SHARED((npad, 128), jnp.float32),
            pltpu.SemaphoreType.DMA,
            pltpu.SemaphoreType.DMA,
            pltpu.SemaphoreType.DMA,
            pltpu.SemaphoreType.DMA,
        ],
    )
    def k(pay_hbm, dst_hbm, zero_hbm, acc_hbm, idx_v0, idx_v1, pay_v, acc_s,
          isem0, psem0, isem1, psem1):
        sid = lax.axis_index("s")
        row0 = sid * rows_per_w
        idxs = (idx_v0, idx_v1)
        isems = (isem0, isem1)
        psems = (psem0, psem1)
        pltpu.sync_copy(zero_hbm, acc_s.at[pl.ds(row0, rows_per_w)])
        plsc.subcore_barrier()

        def issue(ch, b):
            base = sid * per_w + ch * chunk
            pltpu.async_copy(dst_hbm.at[pl.ds(base, chunk)], idxs[b],
                             isems[b])
            pltpu.async_copy(pay_hbm.at[pl.ds(base, chunk)], pay_v.at[b],
                             psems[b])

        def scat(ch, b):
            base = sid * per_w + ch * chunk
            pltpu.make_async_copy(dst_hbm.at[pl.ds(base, chunk)],
                                  idxs[b], isems[b]).wait()
            pltpu.make_async_copy(pay_hbm.at[pl.ds(base, chunk)],
                                  pay_v.at[b], psems[b]).wait()
            pltpu.sync_copy(pay_v.at[b], acc_s.at[idxs[b]], add=True)

        issue(0, 0)
        issue(1, 1)

        @pl.loop(0, nch, step=2)
        def _(i):
            scat(i, 0)

            @pl.when(i + 2 < nch)
            def _():
                issue(i + 2, 0)

            @pl.when(i + 1 < nch)
            def _():
                scat(i + 1, 1)

            @pl.when(i + 3 < nch)
            def _():
                issue(i + 3, 1)

        plsc.subcore_barrier()
        pltpu.sync_copy(acc_s.at[pl.ds(row0, rows_per_w)],
                        acc_hbm.at[pl.ds(row0, rows_per_w)])

    return k(pay, dst, zeros_rows)


def _norm_fc_body(acc_ref, w1_ref, b1_ref, w2_ref, b2_ref, w3_ref, b3_ref,
                  o_ref):
    a = acc_ref[...]
    s = a[:, 64:65] + 1e-16
    node = _gelu(a[:, :64] / s)
    h = _gelu(_dot(node, w1_ref[...]) + b1_ref[...])
    h = _gelu(_dot(h, w2_ref[...]) + b2_ref[...])
    o_ref[...] = _dot(h, w3_ref[...]) + b3_ref[...]


def _norm_fc(acc, p1, p2, p3, blk_n):
    n = acc.shape[0]
    w1, b1 = p1
    w2, b2 = p2
    w3, b3 = p3
    d1, d2, d3 = w1.shape[1], w2.shape[1], w3.shape[1]
    return pl.pallas_call(
        _norm_fc_body,
        grid=(n // blk_n,),
        in_specs=[
            pl.BlockSpec((blk_n, 128), lambda i: (i, 0)),
            pl.BlockSpec((64, d1), lambda i: (0, 0)),
            pl.BlockSpec((1, d1), lambda i: (0, 0)),
            pl.BlockSpec((d1, d2), lambda i: (0, 0)),
            pl.BlockSpec((1, d2), lambda i: (0, 0)),
            pl.BlockSpec((d2, d3), lambda i: (0, 0)),
            pl.BlockSpec((1, d3), lambda i: (0, 0)),
        ],
        out_specs=pl.BlockSpec((blk_n, d3), lambda i: (i, 0)),
        out_shape=jax.ShapeDtypeStruct((n, d3), jnp.float32),
    )(acc, w1, b1[None, :], w2, b2[None, :], w3, b3[None, :])


def kernel(x, edge_index, edge_attr, params):
    n = x.shape[0]
    h = params['gnn1']['W2'].shape[0]
    src = edge_index[0]
    dst = edge_index[1]
    blk_n = 1000
    blk_e = 2000
    npad = ((n + 8 * _NS - 1) // (8 * _NS)) * (8 * _NS)
    zeros_rows = jnp.zeros((npad // _NS, 128), jnp.float32)

    def split_w1(p, in_ch):
        w1 = p['W1']
        wcat = jnp.concatenate([w1[:in_ch], w1[in_ch:2 * in_ch]], axis=1)
        return wcat, w1[2 * in_ch:]

    acc = None
    for i, name in enumerate(('gnn1', 'gnn2', 'gnn3', 'gnn4')):
        p = params[name]
        in_ch = x.shape[1] if i == 0 else h
        wcat, w1e = split_w1(p, in_ch)
        if i == 0:
            ptab = _project(x, wcat, blk_n)
        else:
            ptab = _norm_project(acc, wcat, blk_n)
        gd, gs = _sc_gather(ptab, dst, src)
        pay = _edge_chain(gd, gs, edge_attr, w1e, p['b1'][None, :],
                          p['W2'], p['b2'][None, :],
                          p['W3'], p['b3'][None, :],
                          p['Wa'], p['ba'][None, :], blk_e)
        acc = _sc_scatter(pay, dst, zeros_rows, n)[:n]

    return _norm_fc(acc, params['fc1'], params['fc2'], params['fc3'], blk_n)


# speedup vs baseline: 90.0065x; 1.0339x over previous
import functools

import jax
import jax.numpy as jnp
from jax import lax
from jax.experimental import pallas as pl
from jax.experimental.pallas import tpu as pltpu
from jax.experimental.pallas import tpu_sc as plsc

_HIGH = jax.lax.Precision.DEFAULT

_NC = 2
_NS = 16
_NW = _NC * _NS


def _gelu(v):
    return 0.5 * v * (1.0 + jax.lax.erf(v * 0.7071067811865476))


def _dot(a, b):
    return jax.lax.dot_general(a, b, (((1,), (0,)), ((), ())),
                               precision=_HIGH,
                               preferred_element_type=jnp.float32)


def _proj_body(x_ref, w_ref, p_ref):
    p_ref[...] = _dot(x_ref[...], w_ref[...])


def _project(x, wcat, blk_n):
    n, d = x.shape
    return pl.pallas_call(
        _proj_body,
        grid=(n // blk_n,),
        in_specs=[
            pl.BlockSpec((blk_n, d), lambda i: (i, 0)),
            pl.BlockSpec((d, 128), lambda i: (0, 0)),
        ],
        out_specs=pl.BlockSpec((blk_n, 128), lambda i: (i, 0)),
        out_shape=jax.ShapeDtypeStruct((n, 128), jnp.float32),
    )(x, wcat)


def _norm_proj_body(acc_ref, w_ref, p_ref):
    a = acc_ref[0] + acc_ref[1]
    s = a[:, 64:65] + 1e-16
    node = _gelu(a[:, :64] / s)
    p_ref[...] = _dot(node, w_ref[...])


def _norm_project(acc, wcat, blk_n):
    n = acc.shape[1]
    return pl.pallas_call(
        _norm_proj_body,
        grid=(n // blk_n,),
        in_specs=[
            pl.BlockSpec((2, blk_n, 128), lambda i: (0, i, 0)),
            pl.BlockSpec((64, 128), lambda i: (0, 0)),
        ],
        out_specs=pl.BlockSpec((blk_n, 128), lambda i: (i, 0)),
        out_shape=jax.ShapeDtypeStruct((n, 128), jnp.float32),
    )(acc, wcat)


def _chain_body(gd_ref, gs_ref, ea_ref, w1e_ref, b1_ref, w2_ref, b2_ref,
                w3_ref, b3_ref, wa_ref, ba_ref, pay_ref):
    h0 = gd_ref[:, :64] + gs_ref[:, 64:] + _dot(ea_ref[...], w1e_ref[...]) \
        + b1_ref[...]
    h1 = _gelu(h0)
    h2 = _gelu(_dot(h1, w2_ref[...]) + b2_ref[...])
    h3 = _gelu(_dot(h2, w3_ref[...]) + b3_ref[...])
    r = _dot(h3, wa_ref[...]) + ba_ref[...]
    raw = jnp.where(r >= 0.0, r, 0.1 * r)
    u = jnp.exp(raw)
    blk = h0.shape[0]
    pay_ref[...] = jnp.concatenate(
        [h3 * u, u, jnp.zeros((blk, 63), jnp.float32)], axis=1)


def _edge_chain(gd, gs, ea, w1e, b1, w2, b2, w3, b3, wa, ba, blk_e):
    e = gd.shape[0]
    h = 64
    de = ea.shape[1]
    return pl.pallas_call(
        _chain_body,
        grid=(e // blk_e,),
        in_specs=[
            pl.BlockSpec((blk_e, 128), lambda i: (i, 0)),
            pl.BlockSpec((blk_e, 128), lambda i: (i, 0)),
            pl.BlockSpec((blk_e, de), lambda i: (i, 0)),
            pl.BlockSpec((de, h), lambda i: (0, 0)),
            pl.BlockSpec((1, h), lambda i: (0, 0)),
            pl.BlockSpec((h, h), lambda i: (0, 0)),
            pl.BlockSpec((1, h), lambda i: (0, 0)),
            pl.BlockSpec((h, h), lambda i: (0, 0)),
            pl.BlockSpec((1, h), lambda i: (0, 0)),
            pl.BlockSpec((h, 1), lambda i: (0, 0)),
            pl.BlockSpec((1, 1), lambda i: (0, 0)),
        ],
        out_specs=pl.BlockSpec((blk_e, 128), lambda i: (i, 0)),
        out_shape=jax.ShapeDtypeStruct((e, 128), jnp.float32),
    )(gd, gs, ea, w1e, b1, w2, b2, w3, b3, wa, ba)


def _sc_gather(ptab, dst, src):
    n = ptab.shape[0]
    e = dst.shape[0]
    per_w = e // _NW
    chunk = 200
    nch = per_w // chunk
    mesh = plsc.VectorSubcoreMesh(core_axis_name="c", subcore_axis_name="s")

    @functools.partial(
        pl.kernel,
        mesh=mesh,
        out_type=[
            jax.ShapeDtypeStruct((e, 128), jnp.float32),
            jax.ShapeDtypeStruct((e, 128), jnp.float32),
        ],
        scratch_types=[
            pltpu.VMEM((chunk,), jnp.int32),
            pltpu.VMEM((chunk,), jnp.int32),
            pltpu.VMEM((chunk,), jnp.int32),
            pltpu.VMEM((chunk,), jnp.int32),
            pltpu.VMEM((2, chunk, 128), jnp.float32),
            pltpu.VMEM((2, chunk, 128), jnp.float32),
            pltpu.SemaphoreType.DMA,
            pltpu.SemaphoreType.DMA,
            pltpu.SemaphoreType.DMA,
            pltpu.SemaphoreType.DMA,
            pltpu.SemaphoreType.DMA,
            pltpu.SemaphoreType.DMA,
            pltpu.SemaphoreType.DMA,
            pltpu.SemaphoreType.DMA,
        ],
    )
    def k(p_hbm, dst_hbm, src_hbm, gd_hbm, gs_hbm,
          idx_d0, idx_d1, idx_s0, idx_s1, rows_d, rows_s,
          gsem_d0, gsem_s0, gsem_d1, gsem_s1,
          wsem_d0, wsem_s0, wsem_d1, wsem_s1):
        wid = lax.axis_index("c") * _NS + lax.axis_index("s")
        idxs_d = (idx_d0, idx_d1)
        idxs_s = (idx_s0, idx_s1)
        gsems = ((gsem_d0, gsem_s0), (gsem_d1, gsem_s1))
        wsems = ((wsem_d0, wsem_s0), (wsem_d1, wsem_s1))

        def issue(ch, b):
            base = wid * per_w + ch * chunk
            pltpu.sync_copy(dst_hbm.at[pl.ds(base, chunk)], idxs_d[b])
            pltpu.sync_copy(src_hbm.at[pl.ds(base, chunk)], idxs_s[b])
            pltpu.async_copy(p_hbm.at[idxs_d[b]], rows_d.at[b], gsems[b][0])
            pltpu.async_copy(p_hbm.at[idxs_s[b]], rows_s.at[b], gsems[b][1])

        def gwait(ch, b):
            pltpu.make_async_copy(p_hbm.at[idxs_d[b]], rows_d.at[b],
                                  gsems[b][0]).wait()
            pltpu.make_async_copy(p_hbm.at[idxs_s[b]], rows_s.at[b],
                                  gsems[b][1]).wait()

        def wb(ch, b):
            base = wid * per_w + ch * chunk
            pltpu.async_copy(rows_d.at[b], gd_hbm.at[pl.ds(base, chunk)],
                             wsems[b][0])
            pltpu.async_copy(rows_s.at[b], gs_hbm.at[pl.ds(base, chunk)],
                             wsems[b][1])

        def wbwait(ch, b):
            base = wid * per_w + ch * chunk
            pltpu.make_async_copy(rows_d.at[b], gd_hbm.at[pl.ds(base, chunk)],
                                  wsems[b][0]).wait()
            pltpu.make_async_copy(rows_s.at[b], gs_hbm.at[pl.ds(base, chunk)],
                                  wsems[b][1]).wait()

        issue(0, 0)
        issue(1, 1)

        @pl.loop(0, nch, step=2)
        def _(i):
            gwait(i, 0)
            wb(i, 0)

            @pl.when(i + 2 < nch)
            def _():
                wbwait(i, 0)
                issue(i + 2, 0)

            @pl.when(i + 1 < nch)
            def _():
                gwait(i + 1, 1)
                wb(i + 1, 1)

            @pl.when(i + 3 < nch)
            def _():
                wbwait(i + 1, 1)
                issue(i + 3, 1)

        wbwait(nch - 2, 0)
        wbwait(nch - 1, 1)

    return k(ptab, dst, src)


def _sc_scatter(pay, dst, zeros_rows, n):
    e = dst.shape[0]
    per_w = e // _NW
    chunk = 200
    nch = per_w // chunk
    npad = ((n + 8 * _NS - 1) // (8 * _NS)) * (8 * _NS)
    rows_per_w = npad // _NS
    mesh = plsc.VectorSubcoreMesh(core_axis_name="c", subcore_axis_name="s")

    @functools.partial(
        pl.kernel,
        mesh=mesh,
        out_type=jax.ShapeDtypeStruct((_NC, npad, 128), jnp.float32),
        scratch_types=[
            pltpu.VMEM((chunk,), jnp.int32),
            pltpu.VMEM((chunk, 128), jnp.float32),
            pltpu.VMEM_SHARED((npad, 128), jnp.float32),
            pltpu.SemaphoreType.DMA,
            pltpu.SemaphoreType.DMA,
        ],
    )
    def k(pay_hbm, dst_hbm, zero_hbm, acc_hbm, idx_v, pay_v, acc_s,
          isem, psem):
        c = lax.axis_index("c")
        sid = lax.axis_index("s")
        wid = c * _NS + sid
        row0 = sid * rows_per_w
        pltpu.sync_copy(zero_hbm, acc_s.at[pl.ds(row0, rows_per_w)])
        plsc.subcore_barrier()

        def issue(ch):
            base = wid * per_w + ch * chunk
            pltpu.async_copy(dst_hbm.at[pl.ds(base, chunk)], idx_v, isem)
            pltpu.async_copy(pay_hbm.at[pl.ds(base, chunk)], pay_v, psem)

        def scat(ch):
            base = wid * per_w + ch * chunk
            pltpu.make_async_copy(dst_hbm.at[pl.ds(base, chunk)],
                                  idx_v, isem).wait()
            pltpu.make_async_copy(pay_hbm.at[pl.ds(base, chunk)],
                                  pay_v, psem).wait()
            pltpu.sync_copy(pay_v, acc_s.at[idx_v], add=True)

        issue(0)

        @pl.loop(0, nch)
        def _(i):
            scat(i)

            @pl.when(i + 1 < nch)
            def _():
                issue(i + 1)

        plsc.subcore_barrier()
        pltpu.sync_copy(acc_s.at[pl.ds(row0, rows_per_w)],
                        acc_hbm.at[c].at[pl.ds(row0, rows_per_w)])

    return k(pay, dst, zeros_rows)


def _norm_fc_body(acc_ref, w1_ref, b1_ref, w2_ref, b2_ref, w3_ref, b3_ref,
                  o_ref):
    a = acc_ref[0] + acc_ref[1]
    s = a[:, 64:65] + 1e-16
    node = _gelu(a[:, :64] / s)
    h = _gelu(_dot(node, w1_ref[...]) + b1_ref[...])
    h = _gelu(_dot(h, w2_ref[...]) + b2_ref[...])
    o_ref[...] = _dot(h, w3_ref[...]) + b3_ref[...]


def _norm_fc(acc, p1, p2, p3, blk_n):
    n = acc.shape[1]
    w1, b1 = p1
    w2, b2 = p2
    w3, b3 = p3
    d1, d2, d3 = w1.shape[1], w2.shape[1], w3.shape[1]
    return pl.pallas_call(
        _norm_fc_body,
        grid=(n // blk_n,),
        in_specs=[
            pl.BlockSpec((2, blk_n, 128), lambda i: (0, i, 0)),
            pl.BlockSpec((64, d1), lambda i: (0, 0)),
            pl.BlockSpec((1, d1), lambda i: (0, 0)),
            pl.BlockSpec((d1, d2), lambda i: (0, 0)),
            pl.BlockSpec((1, d2), lambda i: (0, 0)),
            pl.BlockSpec((d2, d3), lambda i: (0, 0)),
            pl.BlockSpec((1, d3), lambda i: (0, 0)),
        ],
        out_specs=pl.BlockSpec((blk_n, d3), lambda i: (i, 0)),
        out_shape=jax.ShapeDtypeStruct((n, d3), jnp.float32),
    )(acc, w1, b1[None, :], w2, b2[None, :], w3, b3[None, :])


def kernel(x, edge_index, edge_attr, params):
    n = x.shape[0]
    h = params['gnn1']['W2'].shape[0]
    src = edge_index[0]
    dst = edge_index[1]
    blk_n = 1000
    blk_e = 2000
    npad = ((n + 8 * _NS - 1) // (8 * _NS)) * (8 * _NS)
    zeros_rows = jnp.zeros((npad // _NS, 128), jnp.float32)

    def split_w1(p, in_ch):
        w1 = p['W1']
        wcat = jnp.concatenate([w1[:in_ch], w1[in_ch:2 * in_ch]], axis=1)
        return wcat, w1[2 * in_ch:]

    acc = None
    for i, name in enumerate(('gnn1', 'gnn2', 'gnn3', 'gnn4')):
        p = params[name]
        in_ch = x.shape[1] if i == 0 else h
        wcat, w1e = split_w1(p, in_ch)
        if i == 0:
            ptab = _project(x, wcat, blk_n)
        else:
            ptab = _norm_project(acc, wcat, blk_n)
        gd, gs = _sc_gather(ptab, dst, src)
        pay = _edge_chain(gd, gs, edge_attr, w1e, p['b1'][None, :],
                          p['W2'], p['b2'][None, :],
                          p['W3'], p['b3'][None, :],
                          p['Wa'], p['ba'][None, :], blk_e)
        acc = _sc_scatter(pay, dst, zeros_rows, n)[:, :n]

    return _norm_fc(acc, params['fc1'], params['fc2'], params['fc3'], blk_n)


# speedup vs baseline: 100.3149x; 1.1145x over previous
import functools

import jax
import jax.numpy as jnp
from jax import lax
from jax.experimental import pallas as pl
from jax.experimental.pallas import tpu as pltpu
from jax.experimental.pallas import tpu_sc as plsc

_HIGH = jax.lax.Precision.DEFAULT

_NC = 2
_NS = 16
_NW = _NC * _NS


def _gelu(v):
    return 0.5 * v * (1.0 + jax.lax.erf(v * 0.7071067811865476))


def _dot(a, b):
    return jax.lax.dot_general(a, b, (((1,), (0,)), ((), ())),
                               precision=_HIGH,
                               preferred_element_type=jnp.float32)


def _proj_body(x_ref, w_ref, p_ref):
    p_ref[...] = _dot(x_ref[...], w_ref[...])


def _project(x, wcat, blk_n):
    n, d = x.shape
    return pl.pallas_call(
        _proj_body,
        grid=(n // blk_n,),
        in_specs=[
            pl.BlockSpec((blk_n, d), lambda i: (i, 0)),
            pl.BlockSpec((d, 128), lambda i: (0, 0)),
        ],
        out_specs=pl.BlockSpec((blk_n, 128), lambda i: (i, 0)),
        out_shape=jax.ShapeDtypeStruct((n, 128), jnp.float32),
    )(x, wcat)


def _norm_proj_body(acc1_ref, acc2_ref, w_ref, p_ref):
    a = acc1_ref[0] + acc1_ref[1] + acc2_ref[0] + acc2_ref[1]
    s = a[:, 64:65] + 1e-16
    node = _gelu(a[:, :64] / s)
    p_ref[...] = _dot(node, w_ref[...])


def _norm_project(acc1, acc2, wcat, blk_n):
    n = acc1.shape[1]
    return pl.pallas_call(
        _norm_proj_body,
        grid=(n // blk_n,),
        in_specs=[
            pl.BlockSpec((2, blk_n, 128), lambda i: (0, i, 0)),
            pl.BlockSpec((2, blk_n, 128), lambda i: (0, i, 0)),
            pl.BlockSpec((64, 128), lambda i: (0, 0)),
        ],
        out_specs=pl.BlockSpec((blk_n, 128), lambda i: (i, 0)),
        out_shape=jax.ShapeDtypeStruct((n, 128), jnp.float32),
    )(acc1, acc2, wcat)


def _chain_body(gd_ref, gs_ref, ea_ref, w1e_ref, b1_ref, w2_ref, b2_ref,
                w3_ref, b3_ref, wa_ref, ba_ref, pay_ref):
    h0 = gd_ref[:, :64] + gs_ref[:, 64:] + _dot(ea_ref[...], w1e_ref[...]) \
        + b1_ref[...]
    h1 = _gelu(h0)
    h2 = _gelu(_dot(h1, w2_ref[...]) + b2_ref[...])
    h3 = _gelu(_dot(h2, w3_ref[...]) + b3_ref[...])
    r = _dot(h3, wa_ref[...]) + ba_ref[...]
    raw = jnp.where(r >= 0.0, r, 0.1 * r)
    u = jnp.exp(raw)
    blk = h0.shape[0]
    pay_ref[...] = jnp.concatenate(
        [h3 * u, u, jnp.zeros((blk, 63), jnp.float32)], axis=1)


def _edge_chain(gd, gs, ea, w1e, b1, w2, b2, w3, b3, wa, ba, blk_e):
    e = gd.shape[0]
    h = 64
    de = ea.shape[1]
    return pl.pallas_call(
        _chain_body,
        grid=(e // blk_e,),
        in_specs=[
            pl.BlockSpec((blk_e, 128), lambda i: (i, 0)),
            pl.BlockSpec((blk_e, 128), lambda i: (i, 0)),
            pl.BlockSpec((blk_e, de), lambda i: (i, 0)),
            pl.BlockSpec((de, h), lambda i: (0, 0)),
            pl.BlockSpec((1, h), lambda i: (0, 0)),
            pl.BlockSpec((h, h), lambda i: (0, 0)),
            pl.BlockSpec((1, h), lambda i: (0, 0)),
            pl.BlockSpec((h, h), lambda i: (0, 0)),
            pl.BlockSpec((1, h), lambda i: (0, 0)),
            pl.BlockSpec((h, 1), lambda i: (0, 0)),
            pl.BlockSpec((1, 1), lambda i: (0, 0)),
        ],
        out_specs=pl.BlockSpec((blk_e, 128), lambda i: (i, 0)),
        out_shape=jax.ShapeDtypeStruct((e, 128), jnp.float32),
    )(gd, gs, ea, w1e, b1, w2, b2, w3, b3, wa, ba)


def _sc_gather(ptab, dst, src):
    n = ptab.shape[0]
    e = dst.shape[0]
    per_w = e // _NW
    chunk = 200
    nch = per_w // chunk
    mesh = plsc.VectorSubcoreMesh(core_axis_name="c", subcore_axis_name="s")

    @functools.partial(
        pl.kernel,
        mesh=mesh,
        out_type=[
            jax.ShapeDtypeStruct((e, 128), jnp.float32),
            jax.ShapeDtypeStruct((e, 128), jnp.float32),
        ],
        scratch_types=[
            pltpu.VMEM((chunk,), jnp.int32),
            pltpu.VMEM((chunk,), jnp.int32),
            pltpu.VMEM((chunk,), jnp.int32),
            pltpu.VMEM((chunk,), jnp.int32),
            pltpu.VMEM((2, chunk, 128), jnp.float32),
            pltpu.VMEM((2, chunk, 128), jnp.float32),
            pltpu.SemaphoreType.DMA,
            pltpu.SemaphoreType.DMA,
            pltpu.SemaphoreType.DMA,
            pltpu.SemaphoreType.DMA,
            pltpu.SemaphoreType.DMA,
            pltpu.SemaphoreType.DMA,
            pltpu.SemaphoreType.DMA,
            pltpu.SemaphoreType.DMA,
        ],
    )
    def k(p_hbm, dst_hbm, src_hbm, gd_hbm, gs_hbm,
          idx_d0, idx_d1, idx_s0, idx_s1, rows_d, rows_s,
          gsem_d0, gsem_s0, gsem_d1, gsem_s1,
          wsem_d0, wsem_s0, wsem_d1, wsem_s1):
        wid = lax.axis_index("c") * _NS + lax.axis_index("s")
        idxs_d = (idx_d0, idx_d1)
        idxs_s = (idx_s0, idx_s1)
        gsems = ((gsem_d0, gsem_s0), (gsem_d1, gsem_s1))
        wsems = ((wsem_d0, wsem_s0), (wsem_d1, wsem_s1))

        def issue(ch, b):
            base = wid * per_w + ch * chunk
            pltpu.sync_copy(dst_hbm.at[pl.ds(base, chunk)], idxs_d[b])
            pltpu.sync_copy(src_hbm.at[pl.ds(base, chunk)], idxs_s[b])
            pltpu.async_copy(p_hbm.at[idxs_d[b]], rows_d.at[b], gsems[b][0])
            pltpu.async_copy(p_hbm.at[idxs_s[b]], rows_s.at[b], gsems[b][1])

        def gwait(ch, b):
            pltpu.make_async_copy(p_hbm.at[idxs_d[b]], rows_d.at[b],
                                  gsems[b][0]).wait()
            pltpu.make_async_copy(p_hbm.at[idxs_s[b]], rows_s.at[b],
                                  gsems[b][1]).wait()

        def wb(ch, b):
            base = wid * per_w + ch * chunk
            pltpu.async_copy(rows_d.at[b], gd_hbm.at[pl.ds(base, chunk)],
                             wsems[b][0])
            pltpu.async_copy(rows_s.at[b], gs_hbm.at[pl.ds(base, chunk)],
                             wsems[b][1])

        def wbwait(ch, b):
            base = wid * per_w + ch * chunk
            pltpu.make_async_copy(rows_d.at[b], gd_hbm.at[pl.ds(base, chunk)],
                                  wsems[b][0]).wait()
            pltpu.make_async_copy(rows_s.at[b], gs_hbm.at[pl.ds(base, chunk)],
                                  wsems[b][1]).wait()

        issue(0, 0)
        issue(1, 1)

        @pl.loop(0, nch, step=2)
        def _(i):
            gwait(i, 0)
            wb(i, 0)

            @pl.when(i + 2 < nch)
            def _():
                wbwait(i, 0)
                issue(i + 2, 0)

            @pl.when(i + 1 < nch)
            def _():
                gwait(i + 1, 1)
                wb(i + 1, 1)

            @pl.when(i + 3 < nch)
            def _():
                wbwait(i + 1, 1)
                issue(i + 3, 1)

        wbwait(nch - 2, 0)
        wbwait(nch - 1, 1)

    return k(ptab, dst, src)


def _sc_scatter(pay, dst, zeros_rows, n):
    e = dst.shape[0]
    per_w = e // _NW
    chunk = 200
    nch = per_w // chunk
    npad = ((n + 8 * _NS - 1) // (8 * _NS)) * (8 * _NS)
    rows_per_w = npad // _NS
    mesh = plsc.VectorSubcoreMesh(core_axis_name="c", subcore_axis_name="s")

    @functools.partial(
        pl.kernel,
        mesh=mesh,
        out_type=jax.ShapeDtypeStruct((_NC, npad, 128), jnp.float32),
        scratch_types=[
            pltpu.VMEM((chunk,), jnp.int32),
            pltpu.VMEM((chunk, 128), jnp.float32),
            pltpu.VMEM_SHARED((npad, 128), jnp.float32),
            pltpu.SemaphoreType.DMA,
            pltpu.SemaphoreType.DMA,
        ],
    )
    def k(pay_hbm, dst_hbm, zero_hbm, acc_hbm, idx_v, pay_v, acc_s,
          isem, psem):
        c = lax.axis_index("c")
        sid = lax.axis_index("s")
        wid = c * _NS + sid
        row0 = sid * rows_per_w
        pltpu.sync_copy(zero_hbm, acc_s.at[pl.ds(row0, rows_per_w)])
        plsc.subcore_barrier()

        def issue(ch):
            base = wid * per_w + ch * chunk
            pltpu.async_copy(dst_hbm.at[pl.ds(base, chunk)], idx_v, isem)
            pltpu.async_copy(pay_hbm.at[pl.ds(base, chunk)], pay_v, psem)

        def scat(ch):
            base = wid * per_w + ch * chunk
            pltpu.make_async_copy(dst_hbm.at[pl.ds(base, chunk)],
                                  idx_v, isem).wait()
            pltpu.make_async_copy(pay_hbm.at[pl.ds(base, chunk)],
                                  pay_v, psem).wait()
            pltpu.sync_copy(pay_v, acc_s.at[idx_v], add=True)

        issue(0)

        @pl.loop(0, nch)
        def _(i):
            scat(i)

            @pl.when(i + 1 < nch)
            def _():
                issue(i + 1)

        plsc.subcore_barrier()
        pltpu.sync_copy(acc_s.at[pl.ds(row0, rows_per_w)],
                        acc_hbm.at[c].at[pl.ds(row0, rows_per_w)])

    return k(pay, dst, zeros_rows)


def _norm_fc_body(acc1_ref, acc2_ref, w1_ref, b1_ref, w2_ref, b2_ref,
                  w3_ref, b3_ref, o_ref):
    a = acc1_ref[0] + acc1_ref[1] + acc2_ref[0] + acc2_ref[1]
    s = a[:, 64:65] + 1e-16
    node = _gelu(a[:, :64] / s)
    h = _gelu(_dot(node, w1_ref[...]) + b1_ref[...])
    h = _gelu(_dot(h, w2_ref[...]) + b2_ref[...])
    o_ref[...] = _dot(h, w3_ref[...]) + b3_ref[...]


def _norm_fc(acc1, acc2, p1, p2, p3, blk_n):
    n = acc1.shape[1]
    w1, b1 = p1
    w2, b2 = p2
    w3, b3 = p3
    d1, d2, d3 = w1.shape[1], w2.shape[1], w3.shape[1]
    return pl.pallas_call(
        _norm_fc_body,
        grid=(n // blk_n,),
        in_specs=[
            pl.BlockSpec((2, blk_n, 128), lambda i: (0, i, 0)),
            pl.BlockSpec((2, blk_n, 128), lambda i: (0, i, 0)),
            pl.BlockSpec((64, d1), lambda i: (0, 0)),
            pl.BlockSpec((1, d1), lambda i: (0, 0)),
            pl.BlockSpec((d1, d2), lambda i: (0, 0)),
            pl.BlockSpec((1, d2), lambda i: (0, 0)),
            pl.BlockSpec((d2, d3), lambda i: (0, 0)),
            pl.BlockSpec((1, d3), lambda i: (0, 0)),
        ],
        out_specs=pl.BlockSpec((blk_n, d3), lambda i: (i, 0)),
        out_shape=jax.ShapeDtypeStruct((n, d3), jnp.float32),
    )(acc1, acc2, w1, b1[None, :], w2, b2[None, :], w3, b3[None, :])


def kernel(x, edge_index, edge_attr, params):
    n = x.shape[0]
    h = params['gnn1']['W2'].shape[0]
    src = edge_index[0]
    dst = edge_index[1]
    blk_n = 1000
    blk_e = 2000
    npad = ((n + 8 * _NS - 1) // (8 * _NS)) * (8 * _NS)
    zeros_rows = jnp.zeros((npad // _NS, 128), jnp.float32)

    def split_w1(p, in_ch):
        w1 = p['W1']
        wcat = jnp.concatenate([w1[:in_ch], w1[in_ch:2 * in_ch]], axis=1)
        return wcat, w1[2 * in_ch:]

    e = dst.shape[0]
    half = e // 2
    halves = ((dst[:half], src[:half], edge_attr[:half]),
              (dst[half:], src[half:], edge_attr[half:]))

    acc1 = acc2 = None
    for i, name in enumerate(('gnn1', 'gnn2', 'gnn3', 'gnn4')):
        p = params[name]
        in_ch = x.shape[1] if i == 0 else h
        wcat, w1e = split_w1(p, in_ch)
        if i == 0:
            ptab = _project(x, wcat, blk_n)
        else:
            ptab = _norm_project(acc1, acc2, wcat, blk_n)
        accs = []
        gathered = [_sc_gather(ptab, d_h, s_h) for d_h, s_h, _ in halves]
        for (d_h, _, ea_h), (gd, gs) in zip(halves, gathered):
            pay = _edge_chain(gd, gs, ea_h, w1e, p['b1'][None, :],
                              p['W2'], p['b2'][None, :],
                              p['W3'], p['b3'][None, :],
                              p['Wa'], p['ba'][None, :], blk_e)
            accs.append(_sc_scatter(pay, d_h, zeros_rows, n)[:, :n])
        acc1, acc2 = accs

    return _norm_fc(acc1, acc2, params['fc1'], params['fc2'], params['fc3'],
                    blk_n)
